# Initial kernel scaffold; baseline (speedup 1.0000x reference)
import jax
import jax.numpy as jnp
from jax.experimental import pallas as pl


def kernel(embed, edge_index, trend):
    raise NotImplementedError("write your pallas kernel here")


# speedup vs baseline: 2.0893x; 2.0893x over previous
import functools

import jax
import jax.numpy as jnp
from jax import lax
from jax.experimental import pallas as pl
from jax.experimental.pallas import tpu as pltpu
from jax.experimental.pallas import tpu_sc as plsc

N_NODES = 10000
N_EDGES = 320000
D_FEAT = 128
NC = 2
NS = 16
DH = D_FEAT // NC
NGRP = DH // 16
N_PAD = 10112
ROWS_PER_SUB = N_PAD // NS
E_PER_SUB = N_EDGES // NS
BLK = 79
NBLK = ROWS_PER_SUB // BLK
CHUNK = 80
N_CHUNKS = E_PER_SUB // CHUNK


def _sc_body(tab, rowi, coli, trend, out, t1, acc1, acc2, b0, b1, rows,
             ridx, cidx, tv, sem):
    c = lax.axis_index("c")
    s = lax.axis_index("s")
    r0 = s * ROWS_PER_SUB
    g0 = c * N_PAD + r0

    def zrow(r, _):
        for j in range(NGRP):
            b0[r, pl.ds(16 * j, 16)] = jnp.zeros((16,), jnp.float32)
        return _
    lax.fori_loop(0, BLK, zrow, None)
    for k in range(NBLK):
        pltpu.sync_copy(b0, acc1.at[pl.ds(r0 + k * BLK, BLK)])
        pltpu.sync_copy(b0, acc2.at[pl.ds(r0 + k * BLK, BLK)])
    plsc.subcore_barrier()

    def layer(src_hbm, acc):
        def chunk_body(i, _):
            e0 = s * E_PER_SUB + i * CHUNK
            pltpu.sync_copy(rowi.at[pl.ds(c * N_EDGES + e0, CHUNK)], ridx)
            pltpu.sync_copy(coli.at[pl.ds(e0, CHUNK)], cidx)
            pltpu.sync_copy(trend.at[pl.ds(e0, CHUNK)], tv)
            pltpu.async_copy(src_hbm.at[ridx], rows, sem).wait()
            def scale(e, _):
                t16 = plsc.load_gather(tv, [jnp.full((16,), e, jnp.int32)])
                for j in range(NGRP):
                    d = pl.ds(16 * j, 16)
                    rows[e, d] = rows[e, d] * t16
                return _
            lax.fori_loop(0, CHUNK, scale, None)
            pltpu.sync_copy(rows, acc.at[cidx], add=True)
            return _
        lax.fori_loop(0, N_CHUNKS, chunk_body, None)

    layer(tab, acc1)
    plsc.subcore_barrier()
    for k in range(NBLK):
        pltpu.sync_copy(acc1.at[pl.ds(r0 + k * BLK, BLK)], b1)
        pltpu.sync_copy(b1, t1.at[pl.ds(g0 + k * BLK, BLK)])
    plsc.subcore_barrier()

    layer(t1, acc2)
    plsc.subcore_barrier()

    third = jnp.full((16,), 1.0 / 3.0, jnp.float32)
    def add1(r, _):
        for j in range(NGRP):
            d = pl.ds(16 * j, 16)
            b0[r, d] = b0[r, d] + b1[r, d]
        return _
    def add2(r, _):
        for j in range(NGRP):
            d = pl.ds(16 * j, 16)
            b0[r, d] = (b0[r, d] + b1[r, d]) * third
        return _
    for k in range(NBLK):
        pltpu.sync_copy(tab.at[pl.ds(g0 + k * BLK, BLK)], b0)
        pltpu.sync_copy(acc1.at[pl.ds(r0 + k * BLK, BLK)], b1)
        lax.fori_loop(0, BLK, add1, None)
        pltpu.sync_copy(acc2.at[pl.ds(r0 + k * BLK, BLK)], b1)
        lax.fori_loop(0, BLK, add2, None)
        pltpu.sync_copy(b0, out.at[pl.ds(g0 + k * BLK, BLK)])


_sc_kernel = functools.partial(
    pl.kernel,
    out_type=jax.ShapeDtypeStruct((NC * N_PAD, DH), jnp.float32),
    mesh=plsc.VectorSubcoreMesh(core_axis_name="c", subcore_axis_name="s"),
    compiler_params=pltpu.CompilerParams(
        needs_layout_passes=False, use_tc_tiling_on_sc=False),
    scratch_types=[
        pltpu.HBM((NC * N_PAD, DH), jnp.float32),
        pltpu.VMEM_SHARED((N_PAD, DH), jnp.float32),
        pltpu.VMEM_SHARED((N_PAD, DH), jnp.float32),
        pltpu.VMEM((BLK, DH), jnp.float32),
        pltpu.VMEM((BLK, DH), jnp.float32),
        pltpu.VMEM((CHUNK, DH), jnp.float32),
        pltpu.VMEM((CHUNK,), jnp.int32),
        pltpu.VMEM((CHUNK,), jnp.int32),
        pltpu.VMEM((CHUNK,), jnp.float32),
        pltpu.SemaphoreType.DMA,
    ],
)(_sc_body)


def kernel(embed, edge_index, trend):
    row = edge_index[0].astype(jnp.int32)
    col = edge_index[1].astype(jnp.int32)
    e_pad = jnp.pad(embed, ((0, N_PAD - N_NODES), (0, 0)))
    tab = e_pad.reshape(N_PAD, NC, DH).transpose(1, 0, 2).reshape(NC * N_PAD, DH)
    rowi = jnp.concatenate([row, row + N_PAD])
    out = _sc_kernel(tab, rowi, col, trend.astype(jnp.float32))
    out = out.reshape(NC, N_PAD, DH).transpose(1, 0, 2).reshape(N_PAD, D_FEAT)
    return out[:N_NODES]


# speedup vs baseline: 5.7433x; 2.7490x over previous
import functools

import jax
import jax.numpy as jnp
from jax import lax
from jax.experimental import pallas as pl
from jax.experimental.pallas import tpu as pltpu
from jax.experimental.pallas import tpu_sc as plsc

N_NODES = 10000
N_EDGES = 320000
D_FEAT = 128
NC = 2
NS = 16
DH = D_FEAT // NC
NGRP = DH // 16
N_PAD = 10112
ROWS_PER_SUB = N_PAD // NS
E_PER_SUB = N_EDGES // NS
BLK = 79
NBLK = ROWS_PER_SUB // BLK
CHUNK = 80
CH_PER_SUB = E_PER_SUB // CHUNK
NCH = 50
N_IBLK = CH_PER_SUB // NCH


def _sc_body(tab, row2d, col2d, tr2d, out, T, A, b0, b1, rows0, rows1,
             rblk, cblk, tvb, sem0, sem1):
    c = lax.axis_index("c")
    s = lax.axis_index("s")
    r0 = s * ROWS_PER_SUB
    g0 = c * N_PAD + r0

    def zrow(r, _):
        for j in range(NGRP):
            b1[r, pl.ds(16 * j, 16)] = jnp.zeros((16,), jnp.float32)
        return _
    lax.fori_loop(0, BLK, zrow, None)
    for k in range(NBLK):
        pltpu.sync_copy(tab.at[pl.ds(g0 + k * BLK, BLK)], b0)
        pltpu.sync_copy(b0, T.at[pl.ds(r0 + k * BLK, BLK)])
        pltpu.sync_copy(b1, A.at[pl.ds(r0 + k * BLK, BLK)])
    plsc.subcore_barrier()

    def layer(src, acc):
        def scale_chunk(jj, buf):
            def scale(e, _):
                t16 = plsc.load_gather(
                    tvb, [jnp.full((16,), jj, jnp.int32),
                          jnp.full((16,), e, jnp.int32)])
                for j in range(NGRP):
                    d = pl.ds(16 * j, 16)
                    buf[e, d] = buf[e, d] * t16
                return _
            lax.fori_loop(0, CHUNK, scale, None, unroll=4)

        def iblk_body(b, _):
            ch0 = s * CH_PER_SUB + b * NCH
            pltpu.sync_copy(row2d.at[pl.ds(ch0, NCH)], rblk)
            pltpu.sync_copy(col2d.at[pl.ds(ch0, NCH)], cblk)
            pltpu.sync_copy(tr2d.at[pl.ds(ch0, NCH)], tvb)
            pltpu.async_copy(src.at[rblk.at[0]], rows0, sem0)
            def pair_body(m, _):
                j0 = 2 * m
                pltpu.async_copy(src.at[rblk.at[j0 + 1]], rows1, sem1)
                pltpu.make_async_copy(src.at[rblk.at[j0]], rows0, sem0).wait()
                scale_chunk(j0, rows0)
                pltpu.sync_copy(rows0, acc.at[cblk.at[j0]], add=True)
                @pl.when(j0 + 2 < NCH)
                def _start_next():
                    pltpu.async_copy(src.at[rblk.at[j0 + 2]], rows0, sem0)
                pltpu.make_async_copy(src.at[rblk.at[j0 + 1]], rows1, sem1).wait()
                scale_chunk(j0 + 1, rows1)
                pltpu.sync_copy(rows1, acc.at[cblk.at[j0 + 1]], add=True)
                return _
            lax.fori_loop(0, NCH // 2, pair_body, None)
            return _
        lax.fori_loop(0, N_IBLK, iblk_body, None)

    layer(T, A)
    plsc.subcore_barrier()
    for k in range(NBLK):
        pltpu.sync_copy(b1, T.at[pl.ds(r0 + k * BLK, BLK)])
    plsc.subcore_barrier()
    layer(A, T)
    plsc.subcore_barrier()

    third = jnp.full((16,), 1.0 / 3.0, jnp.float32)
    def add1(r, _):
        for j in range(NGRP):
            d = pl.ds(16 * j, 16)
            b0[r, d] = b0[r, d] + b1[r, d]
        return _
    def add2(r, _):
        for j in range(NGRP):
            d = pl.ds(16 * j, 16)
            b0[r, d] = (b0[r, d] + b1[r, d]) * third
        return _
    for k in range(NBLK):
        pltpu.sync_copy(tab.at[pl.ds(g0 + k * BLK, BLK)], b0)
        pltpu.sync_copy(A.at[pl.ds(r0 + k * BLK, BLK)], b1)
        lax.fori_loop(0, BLK, add1, None, unroll=4)
        pltpu.sync_copy(T.at[pl.ds(r0 + k * BLK, BLK)], b1)
        lax.fori_loop(0, BLK, add2, None, unroll=4)
        pltpu.sync_copy(b0, out.at[pl.ds(g0 + k * BLK, BLK)])


_sc_kernel = functools.partial(
    pl.kernel,
    out_type=jax.ShapeDtypeStruct((NC * N_PAD, DH), jnp.float32),
    mesh=plsc.VectorSubcoreMesh(core_axis_name="c", subcore_axis_name="s"),
    compiler_params=pltpu.CompilerParams(
        needs_layout_passes=False, use_tc_tiling_on_sc=False),
    scratch_types=[
        pltpu.VMEM_SHARED((N_PAD, DH), jnp.float32),
        pltpu.VMEM_SHARED((N_PAD, DH), jnp.float32),
        pltpu.VMEM((BLK, DH), jnp.float32),
        pltpu.VMEM((BLK, DH), jnp.float32),
        pltpu.VMEM((CHUNK, DH), jnp.float32),
        pltpu.VMEM((CHUNK, DH), jnp.float32),
        pltpu.VMEM((NCH, CHUNK), jnp.int32),
        pltpu.VMEM((NCH, CHUNK), jnp.int32),
        pltpu.VMEM((NCH, CHUNK), jnp.float32),
        pltpu.SemaphoreType.DMA,
        pltpu.SemaphoreType.DMA,
    ],
)(_sc_body)


def kernel(embed, edge_index, trend):
    row = edge_index[0].astype(jnp.int32)
    col = edge_index[1].astype(jnp.int32)
    e_pad = jnp.pad(embed, ((0, N_PAD - N_NODES), (0, 0)))
    tab = e_pad.reshape(N_PAD, NC, DH).transpose(1, 0, 2).reshape(NC * N_PAD, DH)
    row2d = row.reshape(N_EDGES // CHUNK, CHUNK)
    col2d = col.reshape(N_EDGES // CHUNK, CHUNK)
    tr2d = trend.astype(jnp.float32).reshape(N_EDGES // CHUNK, CHUNK)
    out = _sc_kernel(tab, row2d, col2d, tr2d)
    out = out.reshape(NC, N_PAD, DH).transpose(1, 0, 2).reshape(N_PAD, D_FEAT)
    return out[:N_NODES]


# speedup vs baseline: 6.1926x; 1.0782x over previous
import functools

import jax
import jax.numpy as jnp
from jax import lax
from jax.experimental import pallas as pl
from jax.experimental.pallas import tpu as pltpu
from jax.experimental.pallas import tpu_sc as plsc

N_NODES = 10000
N_EDGES = 320000
D_FEAT = 128
NC = 2
NS = 16
DH = D_FEAT // NC
NGRP = DH // 16
N_PAD = 10112
ROWS_PER_SUB = N_PAD // NS
E_PER_SUB = N_EDGES // NS
BLK = 79
NBLK = ROWS_PER_SUB // BLK
CHUNK = 80
CH_PER_SUB = E_PER_SUB // CHUNK
NBUF = 5
NCH = 25
NGROUP = NCH // NBUF
N_IBLK = CH_PER_SUB // NCH


def _sc_body(tab, row2d, col2d, tr2d, out, T, A, b0, b1,
             rows0, rows1, rows2, rows3, rows4,
             rblk, cblk, tvb,
             g0s, g1s, g2s, g3s, g4s, s0s, s1s, s2s, s3s, s4s):
    rows = [rows0, rows1, rows2, rows3, rows4]
    gsem = [g0s, g1s, g2s, g3s, g4s]
    ssem = [s0s, s1s, s2s, s3s, s4s]
    c = lax.axis_index("c")
    s = lax.axis_index("s")
    r0 = s * ROWS_PER_SUB
    g0 = c * N_PAD + r0

    def zrow(r, _):
        for j in range(NGRP):
            b1[r, pl.ds(16 * j, 16)] = jnp.zeros((16,), jnp.float32)
        return _
    lax.fori_loop(0, BLK, zrow, None)
    for k in range(NBLK):
        pltpu.sync_copy(tab.at[pl.ds(g0 + k * BLK, BLK)], b0)
        pltpu.sync_copy(b0, T.at[pl.ds(r0 + k * BLK, BLK)])
        pltpu.sync_copy(b1, A.at[pl.ds(r0 + k * BLK, BLK)])
    plsc.subcore_barrier()

    def layer(src, acc):
        def scale_chunk(jj, buf):
            def scale(e, _):
                t16 = plsc.load_gather(
                    tvb, [jnp.full((16,), jj, jnp.int32),
                          jnp.full((16,), e, jnp.int32)])
                for j in range(NGRP):
                    d = pl.ds(16 * j, 16)
                    buf[e, d] = buf[e, d] * t16
                return _
            lax.fori_loop(0, CHUNK, scale, None, unroll=4)

        def iblk_body(b, _):
            ch0 = s * CH_PER_SUB + b * NCH
            pltpu.sync_copy(row2d.at[pl.ds(ch0, NCH)], rblk)
            pltpu.sync_copy(col2d.at[pl.ds(ch0, NCH)], cblk)
            pltpu.sync_copy(tr2d.at[pl.ds(ch0, NCH)], tvb)
            for k in range(NBUF):
                pltpu.async_copy(src.at[rblk.at[k]], rows[k], gsem[k])
            for g in range(NGROUP):
                for k in range(NBUF):
                    j = g * NBUF + k
                    pltpu.make_async_copy(src.at[rblk.at[j]], rows[k],
                                          gsem[k]).wait()
                    scale_chunk(j, rows[k])
                    pltpu.async_copy(rows[k], acc.at[cblk.at[j]], ssem[k],
                                     add=True)
                for k in range(NBUF):
                    j = g * NBUF + k
                    pltpu.make_async_copy(rows[k], acc.at[cblk.at[j]],
                                          ssem[k]).wait()
                    if g + 1 < NGROUP:
                        pltpu.async_copy(src.at[rblk.at[j + NBUF]], rows[k],
                                         gsem[k])
            return _
        lax.fori_loop(0, N_IBLK, iblk_body, None)

    layer(T, A)
    plsc.subcore_barrier()
    for k in range(NBLK):
        pltpu.sync_copy(b1, T.at[pl.ds(r0 + k * BLK, BLK)])
    plsc.subcore_barrier()
    layer(A, T)
    plsc.subcore_barrier()

    third = jnp.full((16,), 1.0 / 3.0, jnp.float32)
    def add1(r, _):
        for j in range(NGRP):
            d = pl.ds(16 * j, 16)
            b0[r, d] = b0[r, d] + b1[r, d]
        return _
    def add2(r, _):
        for j in range(NGRP):
            d = pl.ds(16 * j, 16)
            b0[r, d] = (b0[r, d] + b1[r, d]) * third
        return _
    for k in range(NBLK):
        pltpu.sync_copy(tab.at[pl.ds(g0 + k * BLK, BLK)], b0)
        pltpu.sync_copy(A.at[pl.ds(r0 + k * BLK, BLK)], b1)
        lax.fori_loop(0, BLK, add1, None, unroll=4)
        pltpu.sync_copy(T.at[pl.ds(r0 + k * BLK, BLK)], b1)
        lax.fori_loop(0, BLK, add2, None, unroll=4)
        pltpu.sync_copy(b0, out.at[pl.ds(g0 + k * BLK, BLK)])


_sc_kernel = functools.partial(
    pl.kernel,
    out_type=jax.ShapeDtypeStruct((NC * N_PAD, DH), jnp.float32),
    mesh=plsc.VectorSubcoreMesh(core_axis_name="c", subcore_axis_name="s"),
    compiler_params=pltpu.CompilerParams(
        needs_layout_passes=False, use_tc_tiling_on_sc=False),
    scratch_types=[
        pltpu.VMEM_SHARED((N_PAD, DH), jnp.float32),
        pltpu.VMEM_SHARED((N_PAD, DH), jnp.float32),
        pltpu.VMEM((BLK, DH), jnp.float32),
        pltpu.VMEM((BLK, DH), jnp.float32),
        pltpu.VMEM((CHUNK, DH), jnp.float32),
        pltpu.VMEM((CHUNK, DH), jnp.float32),
        pltpu.VMEM((CHUNK, DH), jnp.float32),
        pltpu.VMEM((CHUNK, DH), jnp.float32),
        pltpu.VMEM((CHUNK, DH), jnp.float32),
        pltpu.VMEM((NCH, CHUNK), jnp.int32),
        pltpu.VMEM((NCH, CHUNK), jnp.int32),
        pltpu.VMEM((NCH, CHUNK), jnp.float32),
        pltpu.SemaphoreType.DMA,
        pltpu.SemaphoreType.DMA,
        pltpu.SemaphoreType.DMA,
        pltpu.SemaphoreType.DMA,
        pltpu.SemaphoreType.DMA,
        pltpu.SemaphoreType.DMA,
        pltpu.SemaphoreType.DMA,
        pltpu.SemaphoreType.DMA,
        pltpu.SemaphoreType.DMA,
        pltpu.SemaphoreType.DMA,
    ],
)(_sc_body)


def kernel(embed, edge_index, trend):
    row = edge_index[0].astype(jnp.int32)
    col = edge_index[1].astype(jnp.int32)
    e_pad = jnp.pad(embed, ((0, N_PAD - N_NODES), (0, 0)))
    tab = e_pad.reshape(N_PAD, NC, DH).transpose(1, 0, 2).reshape(NC * N_PAD, DH)
    row2d = row.reshape(N_EDGES // CHUNK, CHUNK)
    col2d = col.reshape(N_EDGES // CHUNK, CHUNK)
    tr2d = trend.astype(jnp.float32).reshape(N_EDGES // CHUNK, CHUNK)
    out = _sc_kernel(tab, row2d, col2d, tr2d)
    out = out.reshape(NC, N_PAD, DH).transpose(1, 0, 2).reshape(N_PAD, D_FEAT)
    return out[:N_NODES]
